# Initial kernel scaffold; baseline (speedup 1.0000x reference)
import functools

import jax
import jax.numpy as jnp
from jax import lax
from jax.experimental import pallas as pl
from jax.experimental.pallas import tpu as pltpu
from jax.experimental.pallas import tpu_sc as plsc

N = 10000
NPAD = 10240
E = 320000
D = 128

NC = 2
NS = 16
NW = NC * NS
EPT = E // NW
CHUNK = 80
NCHUNK = EPT // CHUNK
ROWS_W = NPAD // NS
ZROWS = 64

_mesh = plsc.VectorSubcoreMesh(core_axis_name="c", subcore_axis_name="s")


@functools.partial(
    pl.kernel,
    mesh=_mesh,
    out_type=jax.ShapeDtypeStruct((NC * NPAD, 16), jnp.float32),
    scratch_types=[
        pltpu.VMEM((NCHUNK, CHUNK), jnp.int32),
        pltpu.VMEM((CHUNK, 16), jnp.float32),
        pltpu.VMEM((ROWS_W, 16), jnp.float32),
        pltpu.VMEM_SHARED((NPAD, 16), jnp.float32),
    ],
)
def _deg_kernel(dst_hbm, out_hbm, idx_v, ones_v, stage_v, acc_sh):
    cid = lax.axis_index("c")
    sid = lax.axis_index("s")
    wid = cid * NS + sid

    def fill_ones(i, _):
        ones_v[i] = jnp.ones((16,), jnp.float32)
        return ()
    lax.fori_loop(0, CHUNK, fill_ones, ())

    def fill_zero(i, _):
        stage_v[i] = jnp.zeros((16,), jnp.float32)
        return ()
    lax.fori_loop(0, ROWS_W, fill_zero, ())

    pltpu.sync_copy(stage_v, acc_sh.at[pl.ds(sid * ROWS_W, ROWS_W)])
    plsc.subcore_barrier()

    pltpu.sync_copy(dst_hbm.at[wid], idx_v)

    def body(g, _):
        pltpu.sync_copy(ones_v, acc_sh.at[idx_v.at[g]], add=True)
        return ()
    lax.fori_loop(0, NCHUNK, body, ())

    plsc.subcore_barrier()
    pltpu.sync_copy(acc_sh.at[pl.ds(sid * ROWS_W, ROWS_W)], stage_v)
    pltpu.sync_copy(stage_v, out_hbm.at[pl.ds(cid * NPAD + sid * ROWS_W, ROWS_W)])


@functools.partial(
    pl.kernel,
    mesh=_mesh,
    out_type=jax.ShapeDtypeStruct((NC * NPAD, D), jnp.float32),
    scratch_types=[
        pltpu.VMEM((NCHUNK, CHUNK), jnp.int32),
        pltpu.VMEM((NCHUNK, CHUNK), jnp.int32),
        pltpu.VMEM((CHUNK, D), jnp.float32),
        pltpu.VMEM((CHUNK, D), jnp.float32),
        pltpu.VMEM((ZROWS, D), jnp.float32),
        pltpu.VMEM_SHARED((NPAD, D), jnp.float32),
        pltpu.SemaphoreType.DMA,
        pltpu.SemaphoreType.DMA,
    ],
)
def _agg_kernel(h_hbm, src_hbm, dst_hbm, out_hbm,
                src_v, dst_v, buf_a, buf_b, zero_v, acc_sh, sem_a, sem_b):
    cid = lax.axis_index("c")
    sid = lax.axis_index("s")
    wid = cid * NS + sid

    def fill_zero(i, _):
        def inner(j, _):
            zero_v[i, pl.ds(j * 16, 16)] = jnp.zeros((16,), jnp.float32)
            return ()
        lax.fori_loop(0, D // 16, inner, ())
        return ()
    lax.fori_loop(0, ZROWS, fill_zero, ())

    def zcopy(z, _):
        pltpu.sync_copy(zero_v, acc_sh.at[pl.ds(sid * ROWS_W + z * ZROWS, ZROWS)])
        return ()
    lax.fori_loop(0, ROWS_W // ZROWS, zcopy, ())
    plsc.subcore_barrier()

    pltpu.sync_copy(src_hbm.at[wid], src_v)
    pltpu.sync_copy(dst_hbm.at[wid], dst_v)

    pltpu.async_copy(h_hbm.at[src_v.at[0]], buf_a, sem_a)
    npair = (NCHUNK - 1) // 2

    def pair(p, _):
        ga = 2 * p
        gb = 2 * p + 1
        pltpu.make_async_copy(h_hbm.at[src_v.at[ga]], buf_a, sem_a).wait()
        pltpu.async_copy(h_hbm.at[src_v.at[gb]], buf_b, sem_b)
        pltpu.sync_copy(buf_a, acc_sh.at[dst_v.at[ga]], add=True)
        pltpu.make_async_copy(h_hbm.at[src_v.at[gb]], buf_b, sem_b).wait()
        pltpu.async_copy(h_hbm.at[src_v.at[ga + 2]], buf_a, sem_a)
        pltpu.sync_copy(buf_b, acc_sh.at[dst_v.at[gb]], add=True)
        return ()
    lax.fori_loop(0, npair, pair, ())

    pltpu.make_async_copy(h_hbm.at[src_v.at[NCHUNK - 1]], buf_a, sem_a).wait()
    pltpu.sync_copy(buf_a, acc_sh.at[dst_v.at[NCHUNK - 1]], add=True)

    plsc.subcore_barrier()

    def drain(z, _):
        base = sid * ROWS_W + z * ZROWS
        pltpu.sync_copy(acc_sh.at[pl.ds(base, ZROWS)], zero_v)
        pltpu.sync_copy(zero_v, out_hbm.at[pl.ds(cid * NPAD + base, ZROWS)])
        return ()
    lax.fori_loop(0, ROWS_W // ZROWS, drain, ())


BT = 1024


def _dinv_of(deg_ref):
    deg = deg_ref[:, 0:1] + deg_ref[:, 1:2]
    return jnp.where(deg > 0, lax.rsqrt(deg), 0.0)


def _prep_body(deg_ref, x_ref, w_ref, o_ref):
    dinv = _dinv_of(deg_ref)
    h = jnp.dot(x_ref[...], w_ref[...], preferred_element_type=jnp.float32)
    o_ref[...] = h * dinv


def _mid_body(deg_ref, p_ref, hp_ref, b_ref, w_ref, h_out_ref, hn_out_ref):
    dinv = _dinv_of(deg_ref)
    s = p_ref[0] + p_ref[1] + hp_ref[...]
    h = jnp.maximum(s * dinv + b_ref[...], 0.0)
    h_out_ref[...] = h
    hn = jnp.dot(h, w_ref[...], preferred_element_type=jnp.float32)
    hn_out_ref[...] = hn * dinv


def _fin_body(deg_ref, p_ref, hp_ref, b_ref, o_ref):
    dinv = _dinv_of(deg_ref)
    s = p_ref[0] + p_ref[1] + hp_ref[...]
    o_ref[...] = s * dinv + b_ref[...]


_spec_deg = pl.BlockSpec((BT, 2), lambda i: (i, 0))
_spec_rows = pl.BlockSpec((BT, D), lambda i: (i, 0))
_spec_p = pl.BlockSpec((NC, BT, D), lambda i: (0, i, 0))
_spec_w = pl.BlockSpec((D, D), lambda i: (0, 0))
_spec_b = pl.BlockSpec((1, D), lambda i: (0, 0))

_prep = pl.pallas_call(
    _prep_body,
    grid=(NPAD // BT,),
    in_specs=[_spec_deg, _spec_rows, _spec_w],
    out_specs=_spec_rows,
    out_shape=jax.ShapeDtypeStruct((NPAD, D), jnp.float32),
)

_mid = pl.pallas_call(
    _mid_body,
    grid=(NPAD // BT,),
    in_specs=[_spec_deg, _spec_p, _spec_rows, _spec_b, _spec_w],
    out_specs=[_spec_rows, _spec_rows],
    out_shape=[jax.ShapeDtypeStruct((NPAD, D), jnp.float32),
               jax.ShapeDtypeStruct((NPAD, D), jnp.float32)],
)

_fin = pl.pallas_call(
    _fin_body,
    grid=(NPAD // BT,),
    in_specs=[_spec_deg, _spec_p, _spec_rows, _spec_b],
    out_specs=_spec_rows,
    out_shape=jax.ShapeDtypeStruct((NPAD, D), jnp.float32),
)


def kernel(x, edge_index, W1, b1, W2, b2, W3, b3):
    x = x.astype(jnp.float32)
    src3 = edge_index[0].reshape(NW, NCHUNK, CHUNK)
    dst3 = edge_index[1].reshape(NW, NCHUNK, CHUNK)
    xp = jnp.concatenate([x, jnp.zeros((NPAD - N, D), jnp.float32)], axis=0)

    degp = _deg_kernel(dst3)
    deg2 = degp.reshape(NC, NPAD, 16)[:, :, 0].transpose(1, 0)

    h1p = _prep(deg2, xp, W1)
    p1 = _agg_kernel(h1p, src3, dst3).reshape(NC, NPAD, D)
    h1, h2p = _mid(deg2, p1, h1p, b1.reshape(1, D), W2)
    p2 = _agg_kernel(h2p, src3, dst3).reshape(NC, NPAD, D)
    h2, h3p = _mid(deg2, p2, h2p, b2.reshape(1, D), W3)
    p3 = _agg_kernel(h3p, src3, dst3).reshape(NC, NPAD, D)
    y = _fin(deg2, p3, h3p, b3.reshape(1, D))

    return (y[:N], h1[:N], h2[:N])


# speedup vs baseline: 8.4490x; 8.4490x over previous
import functools

import jax
import jax.numpy as jnp
from jax import lax
from jax.experimental import pallas as pl
from jax.experimental.pallas import tpu as pltpu
from jax.experimental.pallas import tpu_sc as plsc

N = 10000
NPAD = 10240
E = 320000
D = 128

NC = 2
NS = 16
NW = NC * NS
CHUNK = 128
NCHUNK = 79
EPT = NCHUNK * CHUNK
E_PAD = NW * EPT
ROWS_W = NPAD // NS
DRAIN = ROWS_W // CHUNK

_mesh = plsc.VectorSubcoreMesh(core_axis_name="c", subcore_axis_name="s")


@functools.partial(
    pl.kernel,
    mesh=_mesh,
    out_type=jax.ShapeDtypeStruct((NC * NPAD, D), jnp.float32),
    scratch_types=[
        pltpu.VMEM((NCHUNK, CHUNK), jnp.int32),
        pltpu.VMEM((NCHUNK, CHUNK), jnp.int32),
        pltpu.VMEM((CHUNK, D), jnp.float32),
        pltpu.VMEM_SHARED((NPAD, D), jnp.float32),
        pltpu.SemaphoreType.DMA,
    ],
)
def _agg_kernel(h_hbm, src_hbm, dst_hbm, out_hbm,
                src_v, dst_v, buf, acc_sh, sem):
    cid = lax.axis_index("c")
    sid = lax.axis_index("s")
    wid = cid * NS + sid

    def fill_zero(i, _):
        def inner(j, _):
            buf[i, pl.ds(j * 16, 16)] = jnp.zeros((16,), jnp.float32)
            return ()
        lax.fori_loop(0, D // 16, inner, ())
        return ()
    lax.fori_loop(0, CHUNK, fill_zero, ())

    def zcopy(z, _):
        pltpu.sync_copy(buf, acc_sh.at[pl.ds(sid * ROWS_W + z * CHUNK, CHUNK)])
        return ()
    lax.fori_loop(0, DRAIN, zcopy, ())
    plsc.subcore_barrier()

    pltpu.sync_copy(src_hbm.at[wid], src_v)
    pltpu.sync_copy(dst_hbm.at[wid], dst_v)

    def body(g, _):
        pltpu.async_copy(h_hbm.at[src_v.at[g]], buf, sem).wait()
        pltpu.sync_copy(buf, acc_sh.at[dst_v.at[g]], add=True)
        return ()
    lax.fori_loop(0, NCHUNK, body, ())

    plsc.subcore_barrier()

    def drain(z, _):
        base = sid * ROWS_W + z * CHUNK
        pltpu.sync_copy(acc_sh.at[pl.ds(base, CHUNK)], buf)
        pltpu.sync_copy(buf, out_hbm.at[pl.ds(cid * NPAD + base, CHUNK)])
        return ()
    lax.fori_loop(0, DRAIN, drain, ())


BT = 1024


def _dinv_of(deg_ref):
    deg = deg_ref[:, 0:1] + deg_ref[:, 1:2] + 1.0
    return lax.rsqrt(deg)


def _prep_body(deg_ref, x_ref, w_ref, o_ref):
    dinv = _dinv_of(deg_ref)
    h = jnp.dot(x_ref[...], w_ref[...], preferred_element_type=jnp.float32)
    o_ref[...] = h * dinv


def _mid_body(deg_ref, p_ref, hp_ref, b_ref, w_ref, h_out_ref, hn_out_ref):
    dinv = _dinv_of(deg_ref)
    s = p_ref[0] + p_ref[1] + hp_ref[...]
    h = jnp.maximum(s * dinv + b_ref[...], 0.0)
    h_out_ref[...] = h
    hn = jnp.dot(h, w_ref[...], preferred_element_type=jnp.float32)
    hn_out_ref[...] = hn * dinv


def _fin_body(deg_ref, p_ref, hp_ref, b_ref, o_ref):
    dinv = _dinv_of(deg_ref)
    s = p_ref[0] + p_ref[1] + hp_ref[...]
    o_ref[...] = s * dinv + b_ref[...]


_spec_deg = pl.BlockSpec((BT, 2), lambda i: (i, 0))
_spec_rows = pl.BlockSpec((BT, D), lambda i: (i, 0))
_spec_p = pl.BlockSpec((NC, BT, D), lambda i: (0, i, 0))
_spec_w = pl.BlockSpec((D, D), lambda i: (0, 0))
_spec_b = pl.BlockSpec((1, D), lambda i: (0, 0))

_prep = pl.pallas_call(
    _prep_body,
    grid=(NPAD // BT,),
    in_specs=[_spec_deg, _spec_rows, _spec_w],
    out_specs=_spec_rows,
    out_shape=jax.ShapeDtypeStruct((NPAD, D), jnp.float32),
)

_mid = pl.pallas_call(
    _mid_body,
    grid=(NPAD // BT,),
    in_specs=[_spec_deg, _spec_p, _spec_rows, _spec_b, _spec_w],
    out_specs=[_spec_rows, _spec_rows],
    out_shape=[jax.ShapeDtypeStruct((NPAD, D), jnp.float32),
               jax.ShapeDtypeStruct((NPAD, D), jnp.float32)],
)

_fin = pl.pallas_call(
    _fin_body,
    grid=(NPAD // BT,),
    in_specs=[_spec_deg, _spec_p, _spec_rows, _spec_b],
    out_specs=_spec_rows,
    out_shape=jax.ShapeDtypeStruct((NPAD, D), jnp.float32),
)


def kernel(x, edge_index, W1, b1, W2, b2, W3, b3):
    x = x.astype(jnp.float32)
    fill = jnp.full((E_PAD - E,), NPAD - 1, dtype=jnp.int32)
    src3 = jnp.concatenate([edge_index[0], fill]).reshape(NW, NCHUNK, CHUNK)
    dst3 = jnp.concatenate([edge_index[1], fill]).reshape(NW, NCHUNK, CHUNK)
    xp = jnp.concatenate([x, jnp.zeros((NPAD - N, D), jnp.float32)], axis=0)

    ones_h = jnp.ones((NPAD, D), jnp.float32)
    pdeg = _agg_kernel(ones_h, src3, dst3).reshape(NC, NPAD, D)
    deg2 = pdeg[:, :, 0].transpose(1, 0)

    h1p = _prep(deg2, xp, W1)
    p1 = _agg_kernel(h1p, src3, dst3).reshape(NC, NPAD, D)
    h1, h2p = _mid(deg2, p1, h1p, b1.reshape(1, D), W2)
    p2 = _agg_kernel(h2p, src3, dst3).reshape(NC, NPAD, D)
    h2, h3p = _mid(deg2, p2, h2p, b2.reshape(1, D), W3)
    p3 = _agg_kernel(h3p, src3, dst3).reshape(NC, NPAD, D)
    y = _fin(deg2, p3, h3p, b3.reshape(1, D))

    return (y[:N], h1[:N], h2[:N])
